# baseline (speedup 1.0000x reference)
import functools

import jax
import jax.numpy as jnp
from jax import lax
from jax.experimental import pallas as pl
from jax.experimental.pallas import tpu as pltpu
from jax.experimental.pallas import tpu_sc as plsc

_NC = 2
_NS = 16
_L = 16
_K = 80


@functools.lru_cache(maxsize=None)
def _sc_agg(n, dh, ch, with_deg):
    mesh = plsc.VectorSubcoreMesh(core_axis_name="c", subcore_axis_name="s")
    out_type = [jax.ShapeDtypeStruct((n, _NC * dh), jnp.float32)]
    if with_deg:
        out_type.append(jax.ShapeDtypeStruct((n, _NC * dh), jnp.float32))

    ns = 5
    scratch = (
        [pltpu.VMEM((ch, _K), jnp.int32)] * 2
        + [pltpu.VMEM((_K, dh), jnp.float32)] * ns
        + ([pltpu.VMEM((_K, _L), jnp.float32)] if with_deg else [])
        + [pltpu.VMEM_SHARED((n, dh), jnp.float32)]
        + ([pltpu.VMEM_SHARED((n, _L), jnp.float32)] if with_deg else [])
        + [pltpu.SemaphoreType.DMA] * ((3 if with_deg else 2) * ns)
    )

    stride = (n // 8 // _NS) * 8
    window = n - (_NS - 1) * stride

    def body(table, edges, zrows, zdeg, *rest):
        if with_deg:
            out_p, out_deg = rest[0], rest[1]
            rest = rest[2:]
        else:
            out_p = rest[0]
            rest = rest[1:]
        srcb, dstb = rest[0], rest[1]
        rows = rest[2:2 + ns]
        rest = rest[2 + ns:]
        if with_deg:
            ones, acc, dacc = rest[0], rest[1], rest[2]
            rest = rest[3:]
        else:
            ones = dacc = None
            acc = rest[0]
            rest = rest[1:]
        gsem = rest[:ns]
        ssem = rest[ns:2 * ns]
        dsem = rest[2 * ns:3 * ns] if with_deg else None

        cid = lax.axis_index("c")
        sid = lax.axis_index("s")
        tab_c = table.at[pl.ds(cid, 2 * n - 1)]

        pltpu.sync_copy(edges.at[0, sid], srcb)
        pltpu.sync_copy(edges.at[1, sid], dstb)

        lo = sid * stride
        pltpu.sync_copy(zrows.at[pl.ds(lo, window)],
                        acc.at[pl.ds(lo, window)])
        if with_deg:
            pltpu.sync_copy(zdeg.at[pl.ds(lo, window)],
                            dacc.at[pl.ds(lo, window)])
            for j in range(_K):
                ones[j, :] = jnp.ones((_L,), jnp.float32)
        plsc.subcore_barrier()

        def g_start(j, ci):
            pltpu.async_copy(tab_c.at[srcb.at[ci]], rows[j], gsem[j])

        def g_wait(j, ci):
            pltpu.make_async_copy(tab_c.at[srcb.at[ci]], rows[j],
                                  gsem[j]).wait()

        def s_start(j, ci):
            pltpu.async_copy(rows[j], acc.at[dstb.at[ci]], ssem[j], add=True)

        def s_wait(j, ci):
            pltpu.make_async_copy(rows[j], acc.at[dstb.at[ci]],
                                  ssem[j]).wait()

        for j in range(ns):
            g_start(j, j)

        def loop(turn, carry):
            base = turn * ns
            for j in range(ns):
                c = base + j
                g_wait(j, c)
                s_start(j, c)
                if with_deg:
                    @pl.when(cid == lax.rem(c, 2))
                    def _():
                        pltpu.async_copy(ones, dacc.at[dstb.at[c]],
                                         dsem[j], add=True)
            for j in range(ns):
                cn = base + ns + j

                @pl.when(cn < ch)
                def _():
                    s_wait(j, cn - ns)
                    g_start(j, cn)

            return carry

        lax.fori_loop(0, ch // ns, loop, 0)
        for j in range(ns):
            s_wait(j, ch - ns + j)
        if with_deg:
            def dloop(turn, carry):
                for j in range(ns):
                    c = turn * ns + j

                    @pl.when(cid == lax.rem(c, 2))
                    def _():
                        pltpu.make_async_copy(ones, dacc.at[dstb.at[c]],
                                              dsem[j]).wait()

                return carry

            lax.fori_loop(0, ch // ns, dloop, 0)

        plsc.subcore_barrier()
        pltpu.sync_copy(acc.at[pl.ds(lo, window)],
                        out_p.at[pl.ds(lo, window), pl.ds(cid * dh, dh)])
        if with_deg:
            pltpu.sync_copy(dacc.at[pl.ds(lo, window)],
                            out_deg.at[pl.ds(lo, window), pl.ds(cid * dh, _L)])

    return pl.kernel(body, mesh=mesh, out_type=out_type,
                     scratch_types=scratch,
                     compiler_params=pltpu.CompilerParams(
                         use_tc_tiling_on_sc=False))


@functools.lru_cache(maxsize=None)
def _tc_combine(n, d, h, relu, block_rows):
    def body(x_ref, p_ref, dg_ref, ws_ref, wn_ref, b_ref, o_ref):
        deg = dg_ref[:, 0:1] + dg_ref[:, d // 2:d // 2 + 1]
        inv = 1.0 / jnp.maximum(deg, 1.0)
        acc = jnp.dot(x_ref[...], ws_ref[...],
                      preferred_element_type=jnp.float32)
        acc += jnp.dot(p_ref[...] * inv, wn_ref[...],
                       preferred_element_type=jnp.float32)
        acc += b_ref[...]
        o_ref[...] = jnp.maximum(acc, 0.0) if relu else acc

    return pl.pallas_call(
        body,
        grid=(n // block_rows,),
        in_specs=[
            pl.BlockSpec((block_rows, d), lambda i: (i, 0)),
            pl.BlockSpec((block_rows, d), lambda i: (i, 0)),
            pl.BlockSpec((block_rows, d), lambda i: (i, 0)),
            pl.BlockSpec((d, h), lambda i: (0, 0)),
            pl.BlockSpec((d, h), lambda i: (0, 0)),
            pl.BlockSpec((1, h), lambda i: (0, 0)),
        ],
        out_specs=pl.BlockSpec((block_rows, h), lambda i: (i, 0)),
        out_shape=jax.ShapeDtypeStruct((n, h), jnp.float32),
    )


def kernel(x, edge_index, W1_self, W1_neigh, b1, W2_self, W2_neigh, b2):
    n, d = x.shape
    e = edge_index.shape[1]
    ch = e // (_NS * _K)
    dh = d // 2
    h1w = W1_self.shape[1]
    h2w = W2_self.shape[1]

    edges = jnp.stack([edge_index[0] * 2, edge_index[1]]).reshape(
        2, _NS, ch, _K)
    zrows = jnp.zeros((n, dh), jnp.float32)
    zdeg = jnp.zeros((n, _L), jnp.float32)

    p1, degt = _sc_agg(n, dh, ch, True)(
        x.reshape(2 * n, dh), edges, zrows, zdeg)
    h1 = _tc_combine(n, d, h1w, True, 1000)(
        x, p1, degt, W1_self, W1_neigh, b1.reshape(1, -1))
    (p2,) = _sc_agg(n, h1w // 2, ch, False)(
        h1.reshape(2 * n, h1w // 2), edges, zrows, zdeg)
    out = _tc_combine(n, h1w, h2w, False, 1000)(
        h1, p2, degt, W2_self, W2_neigh, b2.reshape(1, -1))
    return out


# speedup vs baseline: 1.1115x; 1.1115x over previous
import functools

import jax
import jax.numpy as jnp
from jax import lax
from jax.experimental import pallas as pl
from jax.experimental.pallas import tpu as pltpu
from jax.experimental.pallas import tpu_sc as plsc

_NC = 2
_NS = 16
_L = 16
_K = 80


@functools.lru_cache(maxsize=None)
def _sc_agg(n, dh, ch, with_deg):
    mesh = plsc.VectorSubcoreMesh(core_axis_name="c", subcore_axis_name="s")
    out_type = [jax.ShapeDtypeStruct((n, _NC * dh), jnp.float32)]
    if with_deg:
        out_type.append(jax.ShapeDtypeStruct((n, _NC * dh), jnp.float32))

    ns = 5
    scratch = (
        [pltpu.VMEM((ch, _K), jnp.int32)] * 2
        + [pltpu.VMEM((_K, dh), jnp.float32)] * ns
        + ([pltpu.VMEM((_K, _L), jnp.float32)] if with_deg else [])
        + [pltpu.VMEM_SHARED((n, dh), jnp.float32)]
        + ([pltpu.VMEM_SHARED((n, _L), jnp.float32)] if with_deg else [])
        + [pltpu.SemaphoreType.DMA] * ((3 if with_deg else 2) * ns)
    )

    stride = (n // 8 // _NS) * 8
    window = n - (_NS - 1) * stride

    def body(table, edges, *rest):
        if with_deg:
            out_p, out_deg = rest[0], rest[1]
            rest = rest[2:]
        else:
            out_p = rest[0]
            rest = rest[1:]
        srcb, dstb = rest[0], rest[1]
        rows = rest[2:2 + ns]
        rest = rest[2 + ns:]
        if with_deg:
            ones, acc, dacc = rest[0], rest[1], rest[2]
            rest = rest[3:]
        else:
            ones = dacc = None
            acc = rest[0]
            rest = rest[1:]
        gsem = rest[:ns]
        ssem = rest[ns:2 * ns]
        dsem = rest[2 * ns:3 * ns] if with_deg else None

        cid = lax.axis_index("c")
        sid = lax.axis_index("s")

        pltpu.sync_copy(edges.at[0, sid], srcb)
        pltpu.sync_copy(edges.at[1, sid], dstb)

        def zfill(r, carry):
            for q in range(dh // _L):
                rows[0][r, pl.ds(q * _L, _L)] = jnp.zeros((_L,), jnp.float32)
            return carry

        lax.fori_loop(0, _K, zfill, 0)
        lo = sid * stride
        for k in range(window // _K):
            pltpu.sync_copy(rows[0], acc.at[pl.ds(lo + k * _K, _K)])
        if with_deg:
            for j in range(_K):
                ones[j, :] = jnp.zeros((_L,), jnp.float32)
            for k in range(window // _K):
                pltpu.sync_copy(ones, dacc.at[pl.ds(lo + k * _K, _K)])
            for j in range(_K):
                ones[j, :] = jnp.ones((_L,), jnp.float32)
        plsc.subcore_barrier()

        def xform(ci):
            for q in range(_K // _L):
                v = srcb[ci, pl.ds(q * _L, _L)]
                srcb[ci, pl.ds(q * _L, _L)] = v + v + cid

        def g_start(j, ci):
            pltpu.async_copy(table.at[srcb.at[ci]], rows[j], gsem[j])

        def g_wait(j, ci):
            pltpu.make_async_copy(table.at[srcb.at[ci]], rows[j],
                                  gsem[j]).wait()

        def s_start(j, ci):
            pltpu.async_copy(rows[j], acc.at[dstb.at[ci]], ssem[j], add=True)

        def s_wait(j, ci):
            pltpu.make_async_copy(rows[j], acc.at[dstb.at[ci]],
                                  ssem[j]).wait()

        for j in range(ns):
            xform(j)
            g_start(j, j)

        def loop(turn, carry):
            base = turn * ns
            for j in range(ns):
                c = base + j
                g_wait(j, c)
                s_start(j, c)
                if with_deg:
                    @pl.when(cid == lax.rem(c, 2))
                    def _():
                        pltpu.async_copy(ones, dacc.at[dstb.at[c]],
                                         dsem[j], add=True)
            for j in range(ns):
                cn = base + ns + j

                @pl.when(cn < ch)
                def _():
                    xform(cn)
                    s_wait(j, cn - ns)
                    g_start(j, cn)

            return carry

        lax.fori_loop(0, ch // ns, loop, 0)
        for j in range(ns):
            s_wait(j, ch - ns + j)
        if with_deg:
            def dloop(turn, carry):
                for j in range(ns):
                    c = turn * ns + j

                    @pl.when(cid == lax.rem(c, 2))
                    def _():
                        pltpu.make_async_copy(ones, dacc.at[dstb.at[c]],
                                              dsem[j]).wait()

                return carry

            lax.fori_loop(0, ch // ns, dloop, 0)

        plsc.subcore_barrier()
        pltpu.sync_copy(acc.at[pl.ds(lo, window)],
                        out_p.at[pl.ds(lo, window), pl.ds(cid * dh, dh)])
        if with_deg:
            pltpu.sync_copy(dacc.at[pl.ds(lo, window)],
                            out_deg.at[pl.ds(lo, window), pl.ds(cid * dh, _L)])

    return pl.kernel(body, mesh=mesh, out_type=out_type,
                     scratch_types=scratch,
                     compiler_params=pltpu.CompilerParams(
                         use_tc_tiling_on_sc=False))


@functools.lru_cache(maxsize=None)
def _tc_combine(n, d, h, relu, block_rows):
    def body(x_ref, p_ref, dg_ref, ws_ref, wn_ref, b_ref, o_ref):
        deg = dg_ref[:, 0:1] + dg_ref[:, d // 2:d // 2 + 1]
        inv = 1.0 / jnp.maximum(deg, 1.0)
        acc = jnp.dot(x_ref[...], ws_ref[...],
                      preferred_element_type=jnp.float32)
        acc += jnp.dot(p_ref[...] * inv, wn_ref[...],
                       preferred_element_type=jnp.float32)
        acc += b_ref[...]
        o_ref[...] = jnp.maximum(acc, 0.0) if relu else acc

    return pl.pallas_call(
        body,
        grid=(n // block_rows,),
        in_specs=[
            pl.BlockSpec((block_rows, d), lambda i: (i, 0)),
            pl.BlockSpec((block_rows, d), lambda i: (i, 0)),
            pl.BlockSpec((block_rows, d), lambda i: (i, 0)),
            pl.BlockSpec((d, h), lambda i: (0, 0)),
            pl.BlockSpec((d, h), lambda i: (0, 0)),
            pl.BlockSpec((1, h), lambda i: (0, 0)),
        ],
        out_specs=pl.BlockSpec((block_rows, h), lambda i: (i, 0)),
        out_shape=jax.ShapeDtypeStruct((n, h), jnp.float32),
    )


def kernel(x, edge_index, W1_self, W1_neigh, b1, W2_self, W2_neigh, b2):
    n, d = x.shape
    e = edge_index.shape[1]
    ch = e // (_NS * _K)
    dh = d // 2
    h1w = W1_self.shape[1]
    h2w = W2_self.shape[1]

    edges = edge_index.reshape(2, _NS, ch, _K)

    p1, degt = _sc_agg(n, dh, ch, True)(x.reshape(2 * n, dh), edges)
    h1 = _tc_combine(n, d, h1w, True, 2000)(
        x, p1, degt, W1_self, W1_neigh, b1.reshape(1, -1))
    (p2,) = _sc_agg(n, h1w // 2, ch, False)(
        h1.reshape(2 * n, h1w // 2), edges)
    out = _tc_combine(n, h1w, h2w, False, 2000)(
        h1, p2, degt, W2_self, W2_neigh, b2.reshape(1, -1))
    return out


# speedup vs baseline: 1.1595x; 1.0432x over previous
import functools

import jax
import jax.numpy as jnp
from jax import lax
from jax.experimental import pallas as pl
from jax.experimental.pallas import tpu as pltpu
from jax.experimental.pallas import tpu_sc as plsc

_NC = 2
_NS = 16
_L = 16
_K = 80


@functools.lru_cache(maxsize=None)
def _sc_agg(n, dh, ch, with_deg):
    mesh = plsc.VectorSubcoreMesh(core_axis_name="c", subcore_axis_name="s")
    out_type = [jax.ShapeDtypeStruct((n, _NC * dh), jnp.float32)]
    if with_deg:
        out_type.append(jax.ShapeDtypeStruct((n, _NC * dh), jnp.float32))

    ns = 7 if with_deg else 8
    scratch = (
        [pltpu.VMEM((ch, _K), jnp.int32)] * 2
        + [pltpu.VMEM((_K, dh), jnp.float32)] * ns
        + ([pltpu.VMEM((_K, _L), jnp.float32)] if with_deg else [])
        + [pltpu.VMEM_SHARED((n, dh), jnp.float32)]
        + ([pltpu.VMEM_SHARED((n, _L), jnp.float32)] if with_deg else [])
        + [pltpu.SemaphoreType.DMA] * ((3 if with_deg else 2) * ns)
    )

    stride = (n // 8 // _NS) * 8
    window = n - (_NS - 1) * stride

    def body(table, edges, *rest):
        if with_deg:
            out_p, out_deg = rest[0], rest[1]
            rest = rest[2:]
        else:
            out_p = rest[0]
            rest = rest[1:]
        srcb, dstb = rest[0], rest[1]
        rows = rest[2:2 + ns]
        rest = rest[2 + ns:]
        if with_deg:
            ones, acc, dacc = rest[0], rest[1], rest[2]
            rest = rest[3:]
        else:
            ones = dacc = None
            acc = rest[0]
            rest = rest[1:]
        gsem = rest[:ns]
        ssem = rest[ns:2 * ns]
        dsem = rest[2 * ns:3 * ns] if with_deg else None

        cid = lax.axis_index("c")
        sid = lax.axis_index("s")

        pltpu.sync_copy(edges.at[0, sid], srcb)
        pltpu.sync_copy(edges.at[1, sid], dstb)

        def zfill(r, carry):
            for q in range(dh // _L):
                rows[0][r, pl.ds(q * _L, _L)] = jnp.zeros((_L,), jnp.float32)
            return carry

        lax.fori_loop(0, _K, zfill, 0)
        lo = sid * stride
        for k in range(window // _K):
            pltpu.sync_copy(rows[0], acc.at[pl.ds(lo + k * _K, _K)])
        if with_deg:
            for j in range(_K):
                ones[j, :] = jnp.zeros((_L,), jnp.float32)
            for k in range(window // _K):
                pltpu.sync_copy(ones, dacc.at[pl.ds(lo + k * _K, _K)])
            for j in range(_K):
                ones[j, :] = jnp.ones((_L,), jnp.float32)
        plsc.subcore_barrier()

        def xform(ci):
            for q in range(_K // _L):
                v = srcb[ci, pl.ds(q * _L, _L)]
                srcb[ci, pl.ds(q * _L, _L)] = v + v + cid

        def g_start(j, ci):
            pltpu.async_copy(table.at[srcb.at[ci]], rows[j], gsem[j])

        def g_wait(j, ci):
            pltpu.make_async_copy(table.at[srcb.at[ci]], rows[j],
                                  gsem[j]).wait()

        def s_start(j, ci):
            pltpu.async_copy(rows[j], acc.at[dstb.at[ci]], ssem[j], add=True)

        def s_wait(j, ci):
            pltpu.make_async_copy(rows[j], acc.at[dstb.at[ci]],
                                  ssem[j]).wait()

        for j in range(ns):
            xform(j)
            g_start(j, j)

        def loop(turn, carry):
            base = turn * ns
            for j in range(ns):
                c = base + j

                @pl.when(c < ch)
                def _():
                    g_wait(j, c)
                    s_start(j, c)

                if with_deg:
                    @pl.when((c < ch) & (cid == lax.rem(c, 2)))
                    def _():
                        pltpu.async_copy(ones, dacc.at[dstb.at[c]],
                                         dsem[j], add=True)
            for j in range(ns):
                cn = base + ns + j

                @pl.when(cn < ch)
                def _():
                    xform(cn)
                    s_wait(j, cn - ns)
                    g_start(j, cn)

            return carry

        turns = (ch + ns - 1) // ns
        lax.fori_loop(0, turns, loop, 0)
        for j in range(ns):
            c_last = ch - ns + j
            s_wait(c_last % ns, c_last)
        if with_deg:
            def dloop(turn, carry):
                for j in range(ns):
                    c = turn * ns + j

                    @pl.when((c < ch) & (cid == lax.rem(c, 2)))
                    def _():
                        pltpu.make_async_copy(ones, dacc.at[dstb.at[c]],
                                              dsem[j]).wait()

                return carry

            lax.fori_loop(0, turns, dloop, 0)

        plsc.subcore_barrier()
        pltpu.sync_copy(acc.at[pl.ds(lo, window)],
                        out_p.at[pl.ds(lo, window), pl.ds(cid * dh, dh)])
        if with_deg:
            pltpu.sync_copy(dacc.at[pl.ds(lo, window)],
                            out_deg.at[pl.ds(lo, window), pl.ds(cid * dh, _L)])

    return pl.kernel(body, mesh=mesh, out_type=out_type,
                     scratch_types=scratch,
                     compiler_params=pltpu.CompilerParams(
                         use_tc_tiling_on_sc=False))


@functools.lru_cache(maxsize=None)
def _tc_combine(n, d, h, relu, block_rows):
    def body(x_ref, p_ref, dg_ref, ws_ref, wn_ref, b_ref, o_ref):
        deg = dg_ref[:, 0:1] + dg_ref[:, d // 2:d // 2 + 1]
        inv = 1.0 / jnp.maximum(deg, 1.0)
        acc = jnp.dot(x_ref[...], ws_ref[...],
                      preferred_element_type=jnp.float32)
        acc += jnp.dot(p_ref[...] * inv, wn_ref[...],
                       preferred_element_type=jnp.float32)
        acc += b_ref[...]
        o_ref[...] = jnp.maximum(acc, 0.0) if relu else acc

    return pl.pallas_call(
        body,
        grid=(n // block_rows,),
        in_specs=[
            pl.BlockSpec((block_rows, d), lambda i: (i, 0)),
            pl.BlockSpec((block_rows, d), lambda i: (i, 0)),
            pl.BlockSpec((block_rows, d), lambda i: (i, 0)),
            pl.BlockSpec((d, h), lambda i: (0, 0)),
            pl.BlockSpec((d, h), lambda i: (0, 0)),
            pl.BlockSpec((1, h), lambda i: (0, 0)),
        ],
        out_specs=pl.BlockSpec((block_rows, h), lambda i: (i, 0)),
        out_shape=jax.ShapeDtypeStruct((n, h), jnp.float32),
    )


def kernel(x, edge_index, W1_self, W1_neigh, b1, W2_self, W2_neigh, b2):
    n, d = x.shape
    e = edge_index.shape[1]
    ch = e // (_NS * _K)
    dh = d // 2
    h1w = W1_self.shape[1]
    h2w = W2_self.shape[1]

    edges = edge_index.reshape(2, _NS, ch, _K)

    p1, degt = _sc_agg(n, dh, ch, True)(x.reshape(2 * n, dh), edges)
    h1 = _tc_combine(n, d, h1w, True, 2000)(
        x, p1, degt, W1_self, W1_neigh, b1.reshape(1, -1))
    (p2,) = _sc_agg(n, h1w // 2, ch, False)(
        h1.reshape(2 * n, h1w // 2), edges)
    out = _tc_combine(n, h1w, h2w, False, 2000)(
        h1, p2, degt, W2_self, W2_neigh, b2.reshape(1, -1))
    return out


# speedup vs baseline: 1.1627x; 1.0027x over previous
import functools

import jax
import jax.numpy as jnp
from jax import lax
from jax.experimental import pallas as pl
from jax.experimental.pallas import tpu as pltpu
from jax.experimental.pallas import tpu_sc as plsc

_NC = 2
_NS = 16
_L = 16
_K = 80


@functools.lru_cache(maxsize=None)
def _sc_agg(n, dh, ch, with_deg):
    mesh = plsc.VectorSubcoreMesh(core_axis_name="c", subcore_axis_name="s")
    out_type = [jax.ShapeDtypeStruct((n, _NC * dh), jnp.float32)]
    if with_deg:
        out_type.append(jax.ShapeDtypeStruct((n, _NC * dh), jnp.float32))

    ns = 7 if with_deg else 9
    scratch = (
        [pltpu.VMEM((ch, _K), jnp.int32)] * 2
        + [pltpu.VMEM((_K, dh), jnp.float32)] * ns
        + ([pltpu.VMEM((_K, _L), jnp.float32)] if with_deg else [])
        + [pltpu.VMEM_SHARED((n, dh), jnp.float32)]
        + ([pltpu.VMEM_SHARED((n, _L), jnp.float32)] if with_deg else [])
        + [pltpu.SemaphoreType.DMA] * ((3 if with_deg else 2) * ns)
    )

    stride = (n // 8 // _NS) * 8
    window = n - (_NS - 1) * stride

    def body(table, edges, *rest):
        if with_deg:
            out_p, out_deg = rest[0], rest[1]
            rest = rest[2:]
        else:
            out_p = rest[0]
            rest = rest[1:]
        srcb, dstb = rest[0], rest[1]
        rows = rest[2:2 + ns]
        rest = rest[2 + ns:]
        if with_deg:
            ones, acc, dacc = rest[0], rest[1], rest[2]
            rest = rest[3:]
        else:
            ones = dacc = None
            acc = rest[0]
            rest = rest[1:]
        gsem = rest[:ns]
        ssem = rest[ns:2 * ns]
        dsem = rest[2 * ns:3 * ns] if with_deg else None

        cid = lax.axis_index("c")
        sid = lax.axis_index("s")

        pltpu.sync_copy(edges.at[0, sid], srcb)
        pltpu.sync_copy(edges.at[1, sid], dstb)

        def zfill(r, carry):
            for q in range(dh // _L):
                rows[0][r, pl.ds(q * _L, _L)] = jnp.zeros((_L,), jnp.float32)
            return carry

        lax.fori_loop(0, _K, zfill, 0)
        lo = sid * stride
        for k in range(window // _K):
            pltpu.sync_copy(rows[0], acc.at[pl.ds(lo + k * _K, _K)])
        if with_deg:
            for j in range(_K):
                ones[j, :] = jnp.zeros((_L,), jnp.float32)
            for k in range(window // _K):
                pltpu.sync_copy(ones, dacc.at[pl.ds(lo + k * _K, _K)])
            for j in range(_K):
                ones[j, :] = jnp.ones((_L,), jnp.float32)
        plsc.subcore_barrier()

        def xform(ci):
            for q in range(_K // _L):
                v = srcb[ci, pl.ds(q * _L, _L)]
                srcb[ci, pl.ds(q * _L, _L)] = v + v + cid

        def g_start(j, ci):
            pltpu.async_copy(table.at[srcb.at[ci]], rows[j], gsem[j])

        def g_wait(j, ci):
            pltpu.make_async_copy(table.at[srcb.at[ci]], rows[j],
                                  gsem[j]).wait()

        def s_start(j, ci):
            pltpu.async_copy(rows[j], acc.at[dstb.at[ci]], ssem[j], add=True)

        def s_wait(j, ci):
            pltpu.make_async_copy(rows[j], acc.at[dstb.at[ci]],
                                  ssem[j]).wait()

        for j in range(ns):
            xform(j)
            g_start(j, j)

        def loop(turn, carry):
            base = turn * ns
            for j in range(ns):
                c = base + j

                @pl.when(c < ch)
                def _():
                    g_wait(j, c)
                    s_start(j, c)

                if with_deg:
                    @pl.when((c < ch) & (cid == lax.rem(c, 2)))
                    def _():
                        pltpu.async_copy(ones, dacc.at[dstb.at[c]],
                                         dsem[j], add=True)
            for j in range(ns):
                cn = base + ns + j

                @pl.when(cn < ch)
                def _():
                    xform(cn)
                    s_wait(j, cn - ns)
                    g_start(j, cn)

            return carry

        turns = (ch + ns - 1) // ns
        lax.fori_loop(0, turns, loop, 0)
        for j in range(ns):
            c_last = ch - ns + j
            s_wait(c_last % ns, c_last)
        if with_deg:
            def dloop(turn, carry):
                for j in range(ns):
                    c = turn * ns + j

                    @pl.when((c < ch) & (cid == lax.rem(c, 2)))
                    def _():
                        pltpu.make_async_copy(ones, dacc.at[dstb.at[c]],
                                              dsem[j]).wait()

                return carry

            lax.fori_loop(0, turns, dloop, 0)

        plsc.subcore_barrier()
        pltpu.sync_copy(acc.at[pl.ds(lo, window)],
                        out_p.at[pl.ds(lo, window), pl.ds(cid * dh, dh)])
        if with_deg:
            pltpu.sync_copy(dacc.at[pl.ds(lo, window)],
                            out_deg.at[pl.ds(lo, window), pl.ds(cid * dh, _L)])

    return pl.kernel(body, mesh=mesh, out_type=out_type,
                     scratch_types=scratch,
                     compiler_params=pltpu.CompilerParams(
                         use_tc_tiling_on_sc=False))


@functools.lru_cache(maxsize=None)
def _tc_combine(n, d, h, relu, block_rows):
    def body(x_ref, p_ref, dg_ref, ws_ref, wn_ref, b_ref, o_ref):
        deg = dg_ref[:, 0:1] + dg_ref[:, d // 2:d // 2 + 1]
        inv = 1.0 / jnp.maximum(deg, 1.0)
        acc = jnp.dot(x_ref[...], ws_ref[...],
                      preferred_element_type=jnp.float32)
        acc += jnp.dot(p_ref[...] * inv, wn_ref[...],
                       preferred_element_type=jnp.float32)
        acc += b_ref[...]
        o_ref[...] = jnp.maximum(acc, 0.0) if relu else acc

    return pl.pallas_call(
        body,
        grid=(n // block_rows,),
        in_specs=[
            pl.BlockSpec((block_rows, d), lambda i: (i, 0)),
            pl.BlockSpec((block_rows, d), lambda i: (i, 0)),
            pl.BlockSpec((block_rows, d), lambda i: (i, 0)),
            pl.BlockSpec((d, h), lambda i: (0, 0)),
            pl.BlockSpec((d, h), lambda i: (0, 0)),
            pl.BlockSpec((1, h), lambda i: (0, 0)),
        ],
        out_specs=pl.BlockSpec((block_rows, h), lambda i: (i, 0)),
        out_shape=jax.ShapeDtypeStruct((n, h), jnp.float32),
    )


def kernel(x, edge_index, W1_self, W1_neigh, b1, W2_self, W2_neigh, b2):
    n, d = x.shape
    e = edge_index.shape[1]
    ch = e // (_NS * _K)
    dh = d // 2
    h1w = W1_self.shape[1]
    h2w = W2_self.shape[1]

    edges = edge_index.reshape(2, _NS, ch, _K)

    p1, degt = _sc_agg(n, dh, ch, True)(x.reshape(2 * n, dh), edges)
    h1 = _tc_combine(n, d, h1w, True, 2000)(
        x, p1, degt, W1_self, W1_neigh, b1.reshape(1, -1))
    (p2,) = _sc_agg(n, h1w // 2, ch, False)(
        h1.reshape(2 * n, h1w // 2), edges)
    out = _tc_combine(n, h1w, h2w, False, 2000)(
        h1, p2, degt, W2_self, W2_neigh, b2.reshape(1, -1))
    return out


# speedup vs baseline: 1.1636x; 1.0008x over previous
import functools

import jax
import jax.numpy as jnp
from jax import lax
from jax.experimental import pallas as pl
from jax.experimental.pallas import tpu as pltpu
from jax.experimental.pallas import tpu_sc as plsc

_NC = 2
_NS = 16
_L = 16
_K = 80


@functools.lru_cache(maxsize=None)
def _sc_agg(n, dh, ch, with_deg):
    mesh = plsc.VectorSubcoreMesh(core_axis_name="c", subcore_axis_name="s")
    out_type = [jax.ShapeDtypeStruct((n, _NC * dh), jnp.float32)]
    if with_deg:
        out_type.append(jax.ShapeDtypeStruct((n, _NC * dh), jnp.float32))

    ns = 7 if with_deg else 9
    scratch = (
        [pltpu.VMEM((ch * _K,), jnp.int32)] * 2
        + [pltpu.VMEM((_K, dh), jnp.float32)] * ns
        + ([pltpu.VMEM((_K, _L), jnp.float32)] if with_deg else [])
        + [pltpu.VMEM_SHARED((n, dh), jnp.float32)]
        + ([pltpu.VMEM_SHARED((n, _L), jnp.float32)] if with_deg else [])
        + [pltpu.SemaphoreType.DMA] * ((3 if with_deg else 2) * ns)
    )

    stride = (n // 8 // _NS) * 8
    window = n - (_NS - 1) * stride

    def body(table, edges, *rest):
        if with_deg:
            out_p, out_deg = rest[0], rest[1]
            rest = rest[2:]
        else:
            out_p = rest[0]
            rest = rest[1:]
        srcb, dstb = rest[0], rest[1]
        rows = rest[2:2 + ns]
        rest = rest[2 + ns:]
        if with_deg:
            ones, acc, dacc = rest[0], rest[1], rest[2]
            rest = rest[3:]
        else:
            ones = dacc = None
            acc = rest[0]
            rest = rest[1:]
        gsem = rest[:ns]
        ssem = rest[ns:2 * ns]
        dsem = rest[2 * ns:3 * ns] if with_deg else None

        cid = lax.axis_index("c")
        sid = lax.axis_index("s")

        epw = ch * _K
        pltpu.sync_copy(edges.at[0, pl.ds(sid * epw, epw)], srcb)
        pltpu.sync_copy(edges.at[1, pl.ds(sid * epw, epw)], dstb)

        def zfill(r, carry):
            for q in range(dh // _L):
                rows[0][r, pl.ds(q * _L, _L)] = jnp.zeros((_L,), jnp.float32)
            return carry

        lax.fori_loop(0, _K, zfill, 0)
        lo = sid * stride
        for k in range(window // _K):
            pltpu.sync_copy(rows[0], acc.at[pl.ds(lo + k * _K, _K)])
        if with_deg:
            for j in range(_K):
                ones[j, :] = jnp.zeros((_L,), jnp.float32)
            for k in range(window // _K):
                pltpu.sync_copy(ones, dacc.at[pl.ds(lo + k * _K, _K)])
            for j in range(_K):
                ones[j, :] = jnp.ones((_L,), jnp.float32)
        plsc.subcore_barrier()

        def sidx(ci):
            return srcb.at[pl.ds(ci * _K, _K)]

        def didx(ci):
            return dstb.at[pl.ds(ci * _K, _K)]

        def xform(ci):
            for q in range(_K // _L):
                v = srcb[pl.ds(ci * _K + q * _L, _L)]
                srcb[pl.ds(ci * _K + q * _L, _L)] = v + v + cid

        def g_start(j, ci):
            pltpu.async_copy(table.at[sidx(ci)], rows[j], gsem[j])

        def g_wait(j, ci):
            pltpu.make_async_copy(table.at[sidx(ci)], rows[j],
                                  gsem[j]).wait()

        def s_start(j, ci):
            pltpu.async_copy(rows[j], acc.at[didx(ci)], ssem[j], add=True)

        def s_wait(j, ci):
            pltpu.make_async_copy(rows[j], acc.at[didx(ci)],
                                  ssem[j]).wait()

        for j in range(ns):
            xform(j)
            g_start(j, j)

        def loop(turn, carry):
            base = turn * ns
            for j in range(ns):
                c = base + j

                @pl.when(c < ch)
                def _():
                    g_wait(j, c)
                    s_start(j, c)

                if with_deg:
                    @pl.when((c < ch) & (cid == lax.rem(c, 2)))
                    def _():
                        pltpu.async_copy(ones, dacc.at[didx(c)],
                                         dsem[j], add=True)
            for j in range(ns):
                cn = base + ns + j

                @pl.when(cn < ch)
                def _():
                    xform(cn)
                    s_wait(j, cn - ns)
                    g_start(j, cn)

            return carry

        turns = (ch + ns - 1) // ns
        lax.fori_loop(0, turns, loop, 0)
        for j in range(ns):
            c_last = ch - ns + j
            s_wait(c_last % ns, c_last)
        if with_deg:
            def dloop(turn, carry):
                for j in range(ns):
                    c = turn * ns + j

                    @pl.when((c < ch) & (cid == lax.rem(c, 2)))
                    def _():
                        pltpu.make_async_copy(ones, dacc.at[didx(c)],
                                              dsem[j]).wait()

                return carry

            lax.fori_loop(0, turns, dloop, 0)

        plsc.subcore_barrier()
        pltpu.sync_copy(acc.at[pl.ds(lo, window)],
                        out_p.at[pl.ds(lo, window), pl.ds(cid * dh, dh)])
        if with_deg:
            pltpu.sync_copy(dacc.at[pl.ds(lo, window)],
                            out_deg.at[pl.ds(lo, window), pl.ds(cid * dh, _L)])

    return pl.kernel(body, mesh=mesh, out_type=out_type,
                     scratch_types=scratch,
                     compiler_params=pltpu.CompilerParams(
                         use_tc_tiling_on_sc=False))


@functools.lru_cache(maxsize=None)
def _tc_combine(n, d, h, relu, block_rows):
    def body(x_ref, p_ref, dg_ref, ws_ref, wn_ref, b_ref, o_ref):
        deg = dg_ref[:, 0:1] + dg_ref[:, d // 2:d // 2 + 1]
        inv = 1.0 / jnp.maximum(deg, 1.0)
        acc = jnp.dot(x_ref[...], ws_ref[...],
                      preferred_element_type=jnp.float32)
        acc += jnp.dot(p_ref[...] * inv, wn_ref[...],
                       preferred_element_type=jnp.float32)
        acc += b_ref[...]
        o_ref[...] = jnp.maximum(acc, 0.0) if relu else acc

    return pl.pallas_call(
        body,
        grid=(n // block_rows,),
        in_specs=[
            pl.BlockSpec((block_rows, d), lambda i: (i, 0)),
            pl.BlockSpec((block_rows, d), lambda i: (i, 0)),
            pl.BlockSpec((block_rows, d), lambda i: (i, 0)),
            pl.BlockSpec((d, h), lambda i: (0, 0)),
            pl.BlockSpec((d, h), lambda i: (0, 0)),
            pl.BlockSpec((1, h), lambda i: (0, 0)),
        ],
        out_specs=pl.BlockSpec((block_rows, h), lambda i: (i, 0)),
        out_shape=jax.ShapeDtypeStruct((n, h), jnp.float32),
    )


def kernel(x, edge_index, W1_self, W1_neigh, b1, W2_self, W2_neigh, b2):
    n, d = x.shape
    e = edge_index.shape[1]
    ch = e // (_NS * _K)
    dh = d // 2
    h1w = W1_self.shape[1]
    h2w = W2_self.shape[1]

    p1, degt = _sc_agg(n, dh, ch, True)(x.reshape(2 * n, dh), edge_index)
    h1 = _tc_combine(n, d, h1w, True, 2000)(
        x, p1, degt, W1_self, W1_neigh, b1.reshape(1, -1))
    (p2,) = _sc_agg(n, h1w // 2, ch, False)(
        h1.reshape(2 * n, h1w // 2), edge_index)
    out = _tc_combine(n, h1w, h2w, False, 2000)(
        h1, p2, degt, W2_self, W2_neigh, b2.reshape(1, -1))
    return out
